# baseline (speedup 1.0000x reference)
import functools

import jax
import jax.numpy as jnp
from jax import lax
from jax.experimental import pallas as pl
from jax.experimental.pallas import tpu as pltpu
from jax.experimental.pallas import tpu_sc as plsc

_F32 = jnp.float32
_I32 = jnp.int32

_NC = 2
_NS = 16
_NW = _NC * _NS
_CHUNK = 128
_NBUF = 2


def _dense_body(xq_ref, xt_ref, b2_ref, sqc_ref, dqc_ref, dqr_ref,
                wac_q_ref, wvc_q_ref, wvc_t_ref, bvc_ref,
                wat_ref, bat_ref, wvt_t_ref, wvt_qt_ref, wvt_q_ref, bvt_ref,
                waq_s_ref, waq_d_ref, baq_ref, wvq_ref, bvq_ref,
                xqnew_ref, m_ref, asrc_ref, adst_ref):
  xq = xq_ref[...]
  xt = xt_ref[...]
  nq = xq.shape[0]
  nsub = 1 << nq

  aq = jnp.dot(xq, wac_q_ref[...], preferred_element_type=_F32)
  vq = jnp.dot(xq, wvc_q_ref[...], preferred_element_type=_F32) + bvc_ref[...]
  vt = jnp.dot(xt, wvc_t_ref[...], preferred_element_type=_F32)

  sub_ids = lax.broadcasted_iota(_I32, (nsub, 1), 0)
  neg = jnp.float32(-1e30)
  m = jnp.full((nsub, aq.shape[1]), neg, _F32)
  for q in range(nq):
    bitq = ((sub_ids >> q) & 1) > 0
    m = jnp.where(bitq, jnp.maximum(m, aq[q][None, :]), m)
  denom = jnp.zeros((nsub, aq.shape[1]), _F32)
  cnum = jnp.zeros((nsub, aq.shape[1]), _F32)
  for q in range(nq):
    bitf = (((sub_ids >> q) & 1) > 0).astype(_F32)
    e_q = bitf * jnp.exp(jnp.minimum(aq[q][None, :] - m, 0.0))
    denom = denom + e_q
    cnum = cnum + e_q * vq[q][None, :]
  ctab = cnum / jnp.maximum(denom, 1e-9)

  b2 = b2_ref[...]
  onehot = (lax.broadcasted_iota(_I32, (xt.shape[0], nsub), 1)
            == b2).astype(_F32)
  nonempty = (b2 > 0).astype(_F32)
  xqt = jnp.dot(onehot, ctab, preferred_element_type=_F32) + nonempty * vt
  qrow = jnp.sum(xqt, axis=0, keepdims=True)

  wat = wat_ref[...]
  d = xq.shape[1]
  wat_t, wat_qt, wat_q, wat_d = (wat[0:d], wat[d:2 * d],
                                 wat[2 * d:3 * d], wat[3 * d:4 * d])
  aconst = jnp.dot(qrow, wat_q, preferred_element_type=_F32) + bat_ref[...]
  vconst = jnp.dot(qrow, wvt_q_ref[...], preferred_element_type=_F32) + bvt_ref[...]
  nt_ = xt.shape[0]
  asrc_ref[0:nt_, :] = (jnp.dot(xt, wat_t, preferred_element_type=_F32)
                        + jnp.dot(xqt, wat_qt, preferred_element_type=_F32)
                        + aconst)
  asrc_ref[nt_:, :] = jnp.zeros((8, 1), _F32)
  adst_ref[...] = jnp.dot(xt, wat_d, preferred_element_type=_F32)
  m_ref[0:nt_, :] = (jnp.dot(xt, wvt_t_ref[...], preferred_element_type=_F32)
                     + jnp.dot(xqt, wvt_qt_ref[...], preferred_element_type=_F32)
                     + vconst).astype(jnp.bfloat16)
  m_ref[nt_:, :] = jnp.zeros((8, xt.shape[1]), jnp.bfloat16)

  sqc = sqc_ref[...]
  dqc = dqc_ref[...]
  dqr = dqr_ref[...]
  eq = sqc.shape[0]
  oh_s = (lax.broadcasted_iota(_I32, (eq, nq), 1) == sqc).astype(_F32)
  oh_d = (lax.broadcasted_iota(_I32, (eq, nq), 1) == dqc).astype(_F32)
  oh_dt = (lax.broadcasted_iota(_I32, (nq, eq), 0) == dqr).astype(_F32)
  xs = jnp.dot(oh_s, xq, preferred_element_type=_F32)
  xd = jnp.dot(oh_d, xq, preferred_element_type=_F32)
  logit = (jnp.dot(xs, waq_s_ref[...], preferred_element_type=_F32)
           + jnp.dot(xd, waq_d_ref[...], preferred_element_type=_F32)
           + baq_ref[...])
  att = 1.0 / (1.0 + jnp.exp(-logit))
  vals = jnp.dot(xs, wvq_ref[...], preferred_element_type=_F32) + bvq_ref[...]
  xqnew_ref[...] = jnp.dot(oh_dt, att * vals, preferred_element_type=_F32)


def _stripe_copy(s, src, dst, big, last):
  @pl.when(s < _NS - 1)
  def _():
    pltpu.sync_copy(src.at[pl.ds(s * big, big)], dst.at[pl.ds(s * big, big)])

  @pl.when(s == _NS - 1)
  def _():
    off = (_NS - 1) * big
    pltpu.sync_copy(src.at[pl.ds(off, last)], dst.at[pl.ds(off, last)])


def _sc_edge_body(st_hbm, dt_hbm, asrc_hbm, adst_hbm, m_hbm, zeros_hbm,
                  out_hbm, st_v, dt_v, alpha_v, acc, sems,
                  *, nch0, nch1, nt):
  c = lax.axis_index("c")
  s = lax.axis_index("s")
  wid = c * _NS + s
  nchunks = jnp.where(c == 0, nch0, nch1)
  stripe = -(-nt // (8 * _NS)) * 8
  last_stripe = nt - (_NS - 1) * stripe

  pltpu.sync_copy(st_hbm.at[wid], st_v)
  pltpu.sync_copy(dt_hbm.at[wid], dt_v)

  def phase1(asrc_v, adst_v):
    pltpu.sync_copy(asrc_hbm, asrc_v)
    pltpu.sync_copy(adst_hbm, adst_v)

    def chunk1(j, carry):
      for k in range(_CHUNK // 16):
        sv = st_v[j, pl.ds(k * 16, 16)]
        dv = dt_v[j, pl.ds(k * 16, 16)]
        a1 = plsc.load_gather(asrc_v, [sv])
        a2 = plsc.load_gather(adst_v, [dv])
        alpha_v[j, pl.ds(k * 16, 16)] = 1.0 / (1.0 + jnp.exp(-(a1 + a2)))
      return carry

    lax.fori_loop(0, nchunks, chunk1, 0)

  pl.run_scoped(phase1,
                pltpu.VMEM((nt + 8,), _F32),
                pltpu.VMEM((nt,), _F32))

  _stripe_copy(s, zeros_hbm, acc, stripe, last_stripe)
  plsc.subcore_barrier()

  dw = m_hbm.shape[1]
  d = 2 * dw

  def phase2(rows0_v, rows1_v, scaled_v):
    rows_bufs = (rows0_v, rows1_v)
    bufs = tuple(zip(rows_bufs, sems))
    pltpu.async_copy(m_hbm.at[st_v.at[0]], rows0_v, sems[0])

    def pair(jj, carry):
      for b in range(_NBUF):
        rows_b, sem_b = bufs[b]
        o_rows, o_sem = bufs[1 - b]
        j = _NBUF * jj + b
        jn = jax.lax.rem(j + 1, nchunks)
        pltpu.async_copy(m_hbm.at[st_v.at[jn]], o_rows, o_sem)
        pltpu.make_async_copy(m_hbm.at[st_v.at[j]], rows_b, sem_b).wait()

        def row(r, carry2):
          rsplat = jnp.zeros((16,), _I32) + r
          attr = plsc.load_gather(alpha_v,
                                  [jnp.zeros((16,), _I32) + j, rsplat])
          for cg in range(dw // 16):
            v = rows_b[r, pl.ds(cg * 16, 16)]
            vb = plsc.bitcast(v, jnp.bfloat16)
            a, bb = plsc.unpack(vb, format=plsc.PackFormat.INTERLEAVED,
                                preferred_element_type=_F32)
            cols = lax.iota(_I32, 16) * 2 + (cg * 32)
            plsc.store_scatter(scaled_v, [rsplat, cols], a * attr)
            plsc.store_scatter(scaled_v, [rsplat, cols + 1], bb * attr)
          return carry2

        lax.fori_loop(0, _CHUNK, row, 0)
        pltpu.sync_copy(scaled_v, acc.at[dt_v.at[j]], add=True)
      return carry

    lax.fori_loop(0, nchunks // _NBUF, pair, 0)
    pltpu.make_async_copy(m_hbm.at[st_v.at[0]], rows0_v, sems[0]).wait()

  pl.run_scoped(phase2,
                pltpu.VMEM((_CHUNK, dw), _I32),
                pltpu.VMEM((_CHUNK, dw), _I32),
                pltpu.VMEM((_CHUNK, d), _F32))

  plsc.subcore_barrier()
  _stripe_copy(s, acc, out_hbm.at[c], stripe, last_stripe)


def kernel(Xq, Xt, W_ac, b_ac, W_vc, b_vc, W_at, b_at, W_vt, b_vt,
           W_aq, b_aq, W_vq, b_vq, edge_index_q, edge_index_t, cand_idx):
  nq, d = Xq.shape
  nt = Xt.shape[0]
  et = edge_index_t.shape[1]

  cand = cand_idx.astype(_I32)
  marks = jnp.zeros((nq, nt), _I32).at[
      jnp.arange(nq)[:, None], cand].set(1, mode="drop")
  b_mask = jnp.sum(marks * (1 << jnp.arange(nq, dtype=_I32))[:, None], axis=0)
  b2 = b_mask[:, None].astype(_I32)

  sqc = edge_index_q[0].astype(_I32)[:, None]
  dqc = edge_index_q[1].astype(_I32)[:, None]
  dqr = edge_index_q[1].astype(_I32)[None, :]

  outs = pl.pallas_call(
      _dense_body,
      out_shape=(
          jax.ShapeDtypeStruct((nq, d), _F32),
          jax.ShapeDtypeStruct((nt + 8, d), jnp.bfloat16),
          jax.ShapeDtypeStruct((nt + 8, 1), _F32),
          jax.ShapeDtypeStruct((nt, 1), _F32),
      ),
  )(Xq, Xt, b2, sqc, dqc, dqr,
    W_ac[:d], W_vc[:d], W_vc[d:], b_vc[None, :],
    W_at, b_at[None, :], W_vt[:d], W_vt[d:2 * d], W_vt[2 * d:], b_vt[None, :],
    W_aq[:d], W_aq[d:], b_aq[None, :], W_vq, b_vq[None, :])
  xq_new, m_mat, asrc, adst = outs

  nch0, nch1 = 34, 46
  nchmax = max(nch0, nch1)
  et_pad = _NS * (nch0 + nch1) * _CHUNK
  st = edge_index_t[0].astype(_I32)
  dt = edge_index_t[1].astype(_I32)
  st_p = jnp.pad(st, (0, et_pad - et), constant_values=nt)
  dt_p = jnp.pad(dt, (0, et_pad - et), constant_values=0)
  e0 = _NS * nch0 * _CHUNK

  def _slabs(x):
    p0 = x[:e0].reshape(_NS, nch0, _CHUNK)
    p0 = jnp.pad(p0, ((0, 0), (0, nchmax - nch0), (0, 0)))
    p1 = x[e0:].reshape(_NS, nch1, _CHUNK)
    p1 = jnp.pad(p1, ((0, 0), (0, nchmax - nch1), (0, 0)))
    return jnp.concatenate([p0, p1], axis=0)

  st3 = _slabs(st_p)
  dt3 = _slabs(dt_p)
  m_packed = jax.lax.bitcast_convert_type(
      m_mat.reshape(nt + 8, d // 2, 2), _I32)
  asrc_ext = asrc.reshape(nt + 8)
  adst_flat = adst[:, 0]
  zeros_init = jnp.zeros((nt, d), _F32)

  mesh = plsc.VectorSubcoreMesh(core_axis_name="c", subcore_axis_name="s")
  sc_call = pl.kernel(
      functools.partial(_sc_edge_body, nch0=nch0, nch1=nch1, nt=nt),
      out_type=jax.ShapeDtypeStruct((_NC, nt, d), _F32),
      mesh=mesh,
      compiler_params=pltpu.CompilerParams(needs_layout_passes=False,
                                           use_tc_tiling_on_sc=False),
      scratch_types=[
          pltpu.VMEM((nchmax, _CHUNK), _I32),
          pltpu.VMEM((nchmax, _CHUNK), _I32),
          pltpu.VMEM((nchmax, _CHUNK), _F32),
          pltpu.VMEM_SHARED((nt, d), _F32),
          [pltpu.SemaphoreType.DMA] * _NBUF,
      ],
  )
  partials = sc_call(st3, dt3, asrc_ext, adst_flat, m_packed, zeros_init)

  return xq_new, partials[0] + partials[1]


# speedup vs baseline: 1.0970x; 1.0970x over previous
import functools

import jax
import jax.numpy as jnp
from jax import lax
from jax.experimental import pallas as pl
from jax.experimental.pallas import tpu as pltpu
from jax.experimental.pallas import tpu_sc as plsc

_F32 = jnp.float32
_I32 = jnp.int32

_NC = 2
_NS = 16
_NW = _NC * _NS
_CHUNK = 128
_NBUF = 2


def _dense_body(xq_ref, xt_ref, b2_ref, sqc_ref, dqc_ref, dqr_ref,
                wac_q_ref, wvc_q_ref, wvc_t_ref, bvc_ref,
                wat_ref, bat_ref, wvt_t_ref, wvt_qt_ref, wvt_q_ref, bvt_ref,
                waq_s_ref, waq_d_ref, baq_ref, wvq_ref, bvq_ref,
                xqnew_ref, m_ref, asrc_ref, adst_ref):
  xq = xq_ref[...]
  xt = xt_ref[...]
  nq = xq.shape[0]
  nsub = 1 << nq

  aq = jnp.dot(xq, wac_q_ref[...], preferred_element_type=_F32)
  vq = jnp.dot(xq, wvc_q_ref[...], preferred_element_type=_F32) + bvc_ref[...]
  vt = jnp.dot(xt, wvc_t_ref[...], preferred_element_type=_F32)

  sub_ids = lax.broadcasted_iota(_I32, (nsub, 1), 0)
  neg = jnp.float32(-1e30)
  m = jnp.full((nsub, aq.shape[1]), neg, _F32)
  for q in range(nq):
    bitq = ((sub_ids >> q) & 1) > 0
    m = jnp.where(bitq, jnp.maximum(m, aq[q][None, :]), m)
  denom = jnp.zeros((nsub, aq.shape[1]), _F32)
  cnum = jnp.zeros((nsub, aq.shape[1]), _F32)
  for q in range(nq):
    bitf = (((sub_ids >> q) & 1) > 0).astype(_F32)
    e_q = bitf * jnp.exp(jnp.minimum(aq[q][None, :] - m, 0.0))
    denom = denom + e_q
    cnum = cnum + e_q * vq[q][None, :]
  ctab = cnum / jnp.maximum(denom, 1e-9)

  b2 = b2_ref[...]
  onehot = (lax.broadcasted_iota(_I32, (xt.shape[0], nsub), 1)
            == b2).astype(_F32)
  nonempty = (b2 > 0).astype(_F32)
  xqt = jnp.dot(onehot, ctab, preferred_element_type=_F32) + nonempty * vt
  qrow = jnp.sum(xqt, axis=0, keepdims=True)

  wat = wat_ref[...]
  d = xq.shape[1]
  wat_t, wat_qt, wat_q, wat_d = (wat[0:d], wat[d:2 * d],
                                 wat[2 * d:3 * d], wat[3 * d:4 * d])
  aconst = jnp.dot(qrow, wat_q, preferred_element_type=_F32) + bat_ref[...]
  vconst = jnp.dot(qrow, wvt_q_ref[...], preferred_element_type=_F32) + bvt_ref[...]
  nt_ = xt.shape[0]
  asrc_ref[0:nt_, :] = (jnp.dot(xt, wat_t, preferred_element_type=_F32)
                        + jnp.dot(xqt, wat_qt, preferred_element_type=_F32)
                        + aconst)
  asrc_ref[nt_:, :] = jnp.zeros((8, 1), _F32)
  adst_ref[...] = jnp.dot(xt, wat_d, preferred_element_type=_F32)
  m_ref[0:nt_, :] = (jnp.dot(xt, wvt_t_ref[...], preferred_element_type=_F32)
                     + jnp.dot(xqt, wvt_qt_ref[...], preferred_element_type=_F32)
                     + vconst).astype(jnp.bfloat16)
  m_ref[nt_:, :] = jnp.zeros((8, xt.shape[1]), jnp.bfloat16)

  sqc = sqc_ref[...]
  dqc = dqc_ref[...]
  dqr = dqr_ref[...]
  eq = sqc.shape[0]
  oh_s = (lax.broadcasted_iota(_I32, (eq, nq), 1) == sqc).astype(_F32)
  oh_d = (lax.broadcasted_iota(_I32, (eq, nq), 1) == dqc).astype(_F32)
  oh_dt = (lax.broadcasted_iota(_I32, (nq, eq), 0) == dqr).astype(_F32)
  xs = jnp.dot(oh_s, xq, preferred_element_type=_F32)
  xd = jnp.dot(oh_d, xq, preferred_element_type=_F32)
  logit = (jnp.dot(xs, waq_s_ref[...], preferred_element_type=_F32)
           + jnp.dot(xd, waq_d_ref[...], preferred_element_type=_F32)
           + baq_ref[...])
  att = 1.0 / (1.0 + jnp.exp(-logit))
  vals = jnp.dot(xs, wvq_ref[...], preferred_element_type=_F32) + bvq_ref[...]
  xqnew_ref[...] = jnp.dot(oh_dt, att * vals, preferred_element_type=_F32)


def _stripe_copy(s, src, dst, big, last):
  @pl.when(s < _NS - 1)
  def _():
    pltpu.sync_copy(src.at[pl.ds(s * big, big)], dst.at[pl.ds(s * big, big)])

  @pl.when(s == _NS - 1)
  def _():
    off = (_NS - 1) * big
    pltpu.sync_copy(src.at[pl.ds(off, last)], dst.at[pl.ds(off, last)])


def _sc_edge_body(st_hbm, dt_hbm, asrc_hbm, adst_hbm, m_hbm, zeros_hbm,
                  out_hbm, st_v, dt_v, alpha_v, acc, sems,
                  *, nch0, nch1, nt):
  c = lax.axis_index("c")
  s = lax.axis_index("s")
  wid = c * _NS + s
  nchunks = jnp.where(c == 0, nch0, nch1)
  stripe = -(-nt // (8 * _NS)) * 8
  last_stripe = nt - (_NS - 1) * stripe

  pltpu.sync_copy(st_hbm.at[wid], st_v)
  pltpu.sync_copy(dt_hbm.at[wid], dt_v)

  def phase1(asrc_v, adst_v):
    pltpu.sync_copy(asrc_hbm, asrc_v)
    pltpu.sync_copy(adst_hbm, adst_v)

    def chunk1(j, carry):
      for k in range(_CHUNK // 16):
        sv = st_v[j, pl.ds(k * 16, 16)]
        dv = dt_v[j, pl.ds(k * 16, 16)]
        a1 = plsc.load_gather(asrc_v, [sv])
        a2 = plsc.load_gather(adst_v, [dv])
        alpha_v[j, pl.ds(k * 16, 16)] = 1.0 / (1.0 + jnp.exp(-(a1 + a2)))
      return carry

    lax.fori_loop(0, nchunks, chunk1, 0)

  pl.run_scoped(phase1,
                pltpu.VMEM((nt + 8,), _F32),
                pltpu.VMEM((nt,), _F32))

  _stripe_copy(s, zeros_hbm, acc, stripe, last_stripe)
  plsc.subcore_barrier()

  dw = m_hbm.shape[1]
  d = 2 * dw

  def phase2(rows0_v, rows1_v, scaled_v):
    rows_bufs = (rows0_v, rows1_v)
    bufs = tuple(zip(rows_bufs, sems))
    pltpu.async_copy(m_hbm.at[st_v.at[0]], rows0_v, sems[0])

    def pair(jj, carry):
      for b in range(_NBUF):
        rows_b, sem_b = bufs[b]
        o_rows, o_sem = bufs[1 - b]
        j = _NBUF * jj + b
        jn = jax.lax.rem(j + 1, nchunks)
        pltpu.async_copy(m_hbm.at[st_v.at[jn]], o_rows, o_sem)
        pltpu.make_async_copy(m_hbm.at[st_v.at[j]], rows_b, sem_b).wait()

        def row(r, carry2):
          rsplat = jnp.zeros((16,), _I32) + r
          attr = plsc.load_gather(alpha_v,
                                  [jnp.zeros((16,), _I32) + j, rsplat])
          for cg in range(dw // 16):
            v = rows_b[r, pl.ds(cg * 16, 16)]
            vb = plsc.bitcast(v, jnp.bfloat16)
            a, bb = plsc.unpack(vb, format=plsc.PackFormat.INTERLEAVED,
                                preferred_element_type=_F32)
            cols = lax.iota(_I32, 16) * 2 + (cg * 32)
            plsc.store_scatter(scaled_v, [rsplat, cols], a * attr)
            plsc.store_scatter(scaled_v, [rsplat, cols + 1], bb * attr)
          return carry2

        lax.fori_loop(0, _CHUNK, row, 0)
        pltpu.sync_copy(scaled_v, acc.at[dt_v.at[j]], add=True)
      return carry

    lax.fori_loop(0, nchunks // _NBUF, pair, 0)
    pltpu.make_async_copy(m_hbm.at[st_v.at[0]], rows0_v, sems[0]).wait()

  pl.run_scoped(phase2,
                pltpu.VMEM((_CHUNK, dw), _I32),
                pltpu.VMEM((_CHUNK, dw), _I32),
                pltpu.VMEM((_CHUNK, d), _F32))

  plsc.subcore_barrier()
  _stripe_copy(s, acc, out_hbm.at[c], stripe, last_stripe)


def kernel(Xq, Xt, W_ac, b_ac, W_vc, b_vc, W_at, b_at, W_vt, b_vt,
           W_aq, b_aq, W_vq, b_vq, edge_index_q, edge_index_t, cand_idx):
  nq, d = Xq.shape
  nt = Xt.shape[0]
  et = edge_index_t.shape[1]

  cand = cand_idx.astype(_I32)
  marks = jnp.zeros((nq, nt), _I32).at[
      jnp.arange(nq)[:, None], cand].set(1, mode="drop")
  b_mask = jnp.sum(marks * (1 << jnp.arange(nq, dtype=_I32))[:, None], axis=0)
  b2 = b_mask[:, None].astype(_I32)

  sqc = edge_index_q[0].astype(_I32)[:, None]
  dqc = edge_index_q[1].astype(_I32)[:, None]
  dqr = edge_index_q[1].astype(_I32)[None, :]

  outs = pl.pallas_call(
      _dense_body,
      out_shape=(
          jax.ShapeDtypeStruct((nq, d), _F32),
          jax.ShapeDtypeStruct((nt + 8, d), jnp.bfloat16),
          jax.ShapeDtypeStruct((nt + 8, 1), _F32),
          jax.ShapeDtypeStruct((nt, 1), _F32),
      ),
  )(Xq, Xt, b2, sqc, dqc, dqr,
    W_ac[:d], W_vc[:d], W_vc[d:], b_vc[None, :],
    W_at, b_at[None, :], W_vt[:d], W_vt[d:2 * d], W_vt[2 * d:], b_vt[None, :],
    W_aq[:d], W_aq[d:], b_aq[None, :], W_vq, b_vq[None, :])
  xq_new, m_mat, asrc, adst = outs

  nch0, nch1 = 46, 34
  nchmax = max(nch0, nch1)
  et_pad = _NS * (nch0 + nch1) * _CHUNK
  st = edge_index_t[0].astype(_I32)
  dt = edge_index_t[1].astype(_I32)
  st_p = jnp.pad(st, (0, et_pad - et), constant_values=nt)
  dt_p = jnp.pad(dt, (0, et_pad - et), constant_values=0)
  e0 = _NS * nch0 * _CHUNK

  def _slabs(x):
    p0 = x[:e0].reshape(_NS, nch0, _CHUNK)
    p0 = jnp.pad(p0, ((0, 0), (0, nchmax - nch0), (0, 0)))
    p1 = x[e0:].reshape(_NS, nch1, _CHUNK)
    p1 = jnp.pad(p1, ((0, 0), (0, nchmax - nch1), (0, 0)))
    return jnp.concatenate([p0, p1], axis=0)

  st3 = _slabs(st_p)
  dt3 = _slabs(dt_p)
  m_packed = jax.lax.bitcast_convert_type(
      m_mat.reshape(nt + 8, d // 2, 2), _I32)
  asrc_ext = asrc.reshape(nt + 8)
  adst_flat = adst[:, 0]
  zeros_init = jnp.zeros((nt, d), _F32)

  mesh = plsc.VectorSubcoreMesh(core_axis_name="c", subcore_axis_name="s")
  sc_call = pl.kernel(
      functools.partial(_sc_edge_body, nch0=nch0, nch1=nch1, nt=nt),
      out_type=jax.ShapeDtypeStruct((_NC, nt, d), _F32),
      mesh=mesh,
      compiler_params=pltpu.CompilerParams(needs_layout_passes=False,
                                           use_tc_tiling_on_sc=False),
      scratch_types=[
          pltpu.VMEM((nchmax, _CHUNK), _I32),
          pltpu.VMEM((nchmax, _CHUNK), _I32),
          pltpu.VMEM((nchmax, _CHUNK), _F32),
          pltpu.VMEM_SHARED((nt, d), _F32),
          [pltpu.SemaphoreType.DMA] * _NBUF,
      ],
  )
  partials = sc_call(st3, dt3, asrc_ext, adst_flat, m_packed, zeros_init)

  return xq_new, partials[0] + partials[1]
